# baseline (speedup 1.0000x reference)
import functools

import jax
import jax.numpy as jnp
from jax import lax
from jax.experimental import pallas as pl
from jax.experimental.pallas import tpu as pltpu
from jax.experimental.pallas import tpu_sc as plsc

N = 10000
E = 320000
D = 128
B = 4096
R_FD = 2.0
T_FD = 1.0

NC = 2
NS = 16
NW = NC * NS

NPAD = 10240
ROWS_PER_TILE = NPAD // NS
EPW = 10240
EPAD = EPW * NW
CHUNK = 128
NCHUNK = EPW // CHUNK
HCHUNK = NCHUNK // 2
PPW = B // NW
STG = 128

RB = 1024
DB = 512


def _expmap0_proj(u):
    nrm = jnp.maximum(jnp.sqrt(jnp.sum(u * u, axis=1, keepdims=True)), 1e-6)
    x = jnp.tanh(nrm) * u / nrm
    n2 = jnp.maximum(jnp.sqrt(jnp.sum(x * x, axis=1, keepdims=True)), 1e-6)
    return x * jnp.minimum(1.0, (1.0 - 1e-5) / n2)


def _logmap0(x):
    nrm = jnp.maximum(jnp.sqrt(jnp.sum(x * x, axis=1, keepdims=True)), 1e-6)
    p = jnp.clip(nrm, 1e-6, 1.0 - 1e-5)
    return 0.5 * jnp.log((1.0 + p) / (1.0 - p)) * x / nrm


def _stage1_body(x_ref, w_ref, b_ref, o_ref):
    xh = _expmap0_proj(x_ref[...])
    h = _logmap0(xh)
    o_ref[...] = (
        jnp.dot(h, w_ref[...], preferred_element_type=jnp.float32) + b_ref[...]
    )


def _stage2_body(part_ref, degp_ref, w_ref, b_ref, h1_ref, m2_ref):
    deg = jnp.maximum(degp_ref[0] + degp_ref[1], 1.0)
    agg = (part_ref[0] + part_ref[1]) / deg
    agg = jnp.maximum(agg, 0.0)
    h1 = _expmap0_proj(agg)
    h1_ref[...] = h1
    h = _logmap0(h1)
    m2_ref[...] = (
        jnp.dot(h, w_ref[...], preferred_element_type=jnp.float32) + b_ref[...]
    )


def _stage3_body(part_ref, degp_ref, h2_ref):
    deg = jnp.maximum(degp_ref[0] + degp_ref[1], 1.0)
    agg = (part_ref[0] + part_ref[1]) / deg
    h2_ref[...] = _expmap0_proj(agg)


def _stage4_body(l1_ref, r1_ref, l2_ref, r2_ref, o_ref):
    a = l1_ref[...] - r1_ref[...]
    b = l2_ref[...] - r2_ref[...]
    dist = jnp.sum(a * a + b * b, axis=1, keepdims=True)
    o_ref[...] = 1.0 / (jnp.exp((dist - R_FD) / T_FD) + 1.0)


def _stage1(xpad, W1, b1):
    return pl.pallas_call(
        _stage1_body,
        grid=(NPAD // RB,),
        in_specs=[
            pl.BlockSpec((RB, D), lambda i: (i, 0)),
            pl.BlockSpec((D, D), lambda i: (0, 0)),
            pl.BlockSpec((1, D), lambda i: (0, 0)),
        ],
        out_specs=pl.BlockSpec((RB, D), lambda i: (i, 0)),
        out_shape=jax.ShapeDtypeStruct((NPAD, D), jnp.float32),
    )(xpad, W1, b1)


def _stage2(part1, degp3, W2, b2):
    return pl.pallas_call(
        _stage2_body,
        grid=(NPAD // RB,),
        in_specs=[
            pl.BlockSpec((NC, RB, D), lambda i: (0, i, 0)),
            pl.BlockSpec((NC, RB, 1), lambda i: (0, i, 0)),
            pl.BlockSpec((D, D), lambda i: (0, 0)),
            pl.BlockSpec((1, D), lambda i: (0, 0)),
        ],
        out_specs=[
            pl.BlockSpec((RB, D), lambda i: (i, 0)),
            pl.BlockSpec((RB, D), lambda i: (i, 0)),
        ],
        out_shape=[
            jax.ShapeDtypeStruct((NPAD, D), jnp.float32),
            jax.ShapeDtypeStruct((NPAD, D), jnp.float32),
        ],
    )(part1, degp3, W2, b2)


def _stage3(part2, degp3):
    return pl.pallas_call(
        _stage3_body,
        grid=(NPAD // RB,),
        in_specs=[
            pl.BlockSpec((NC, RB, D), lambda i: (0, i, 0)),
            pl.BlockSpec((NC, RB, 1), lambda i: (0, i, 0)),
        ],
        out_specs=pl.BlockSpec((RB, D), lambda i: (i, 0)),
        out_shape=jax.ShapeDtypeStruct((NPAD, D), jnp.float32),
    )(part2, degp3)


def _stage4(L1, R1, L2, R2):
    return pl.pallas_call(
        _stage4_body,
        grid=(B // DB,),
        in_specs=[pl.BlockSpec((DB, D), lambda i: (i, 0))] * 4,
        out_specs=pl.BlockSpec((DB, 1), lambda i: (i, 0)),
        out_shape=jax.ShapeDtypeStruct((B, 1), jnp.float32),
    )(L1, R1, L2, R2)


def _sc_mesh():
    return plsc.VectorSubcoreMesh(core_axis_name="c", subcore_axis_name="s")


def _edge_loop(m, src, dst, acc, srcv, dstv, rows0, rows1, sem0, sem1,
               wid, scatter_extra):
    for h in range(NCHUNK // HCHUNK):
        row0 = wid * NCHUNK + h * HCHUNK
        pltpu.sync_copy(src.at[pl.ds(row0, HCHUNK), :], srcv)
        pltpu.sync_copy(dst.at[pl.ds(row0, HCHUNK), :], dstv)
        pltpu.async_copy(m.at[srcv.at[0]], rows0, sem0)

        def step(t, carry):
            j = 2 * t
            pltpu.make_async_copy(m.at[srcv.at[j]], rows0, sem0).wait()
            pltpu.async_copy(m.at[srcv.at[j + 1]], rows1, sem1)
            pltpu.sync_copy(rows0, acc.at[dstv.at[j]], add=True)
            scatter_extra(dstv, j)
            pltpu.make_async_copy(m.at[srcv.at[j + 1]], rows1, sem1).wait()

            @pl.when(j + 2 < HCHUNK)
            def _():
                pltpu.async_copy(m.at[srcv.at[j + 2]], rows0, sem0)

            pltpu.sync_copy(rows1, acc.at[dstv.at[j + 1]], add=True)
            scatter_extra(dstv, j + 1)
            return carry

        lax.fori_loop(0, HCHUNK // 2, step, 0)


def _agg_deg_body(m, src, dst, zrows, dzer, ones, part, degp,
                  srcv, dstv, rows0, rows1, sem0, sem1, acc,
                  onesv, dstg, dega):
    c = lax.axis_index("c")
    s = lax.axis_index("s")
    wid = s * NC + c
    r0 = s * ROWS_PER_TILE
    pltpu.sync_copy(zrows, rows0)
    for t in range(ROWS_PER_TILE // STG):
        pltpu.sync_copy(rows0, acc.at[pl.ds(r0 + t * STG, STG), :])
    pltpu.sync_copy(dzer, dstg)
    pltpu.sync_copy(dstg, dega.at[pl.ds(r0, ROWS_PER_TILE)])
    pltpu.sync_copy(ones, onesv)
    plsc.subcore_barrier()

    def extra(dv, j):
        pltpu.sync_copy(onesv, dega.at[dv.at[j]], add=True)

    _edge_loop(m, src, dst, acc, srcv, dstv, rows0, rows1, sem0, sem1,
               wid, extra)
    plsc.subcore_barrier()
    for t in range(ROWS_PER_TILE // STG):
        sl = pl.ds(r0 + t * STG, STG)
        pltpu.sync_copy(acc.at[sl, :], rows0)
        pltpu.sync_copy(rows0, part.at[c, sl, :])
    pltpu.sync_copy(dega.at[pl.ds(r0, ROWS_PER_TILE)], dstg)
    pltpu.sync_copy(dstg, degp.at[c, pl.ds(r0, ROWS_PER_TILE)])


def _agg_body(m, src, dst, zrows, part, srcv, dstv, rows0, rows1,
              sem0, sem1, acc):
    c = lax.axis_index("c")
    s = lax.axis_index("s")
    wid = s * NC + c
    r0 = s * ROWS_PER_TILE
    pltpu.sync_copy(zrows, rows0)
    for t in range(ROWS_PER_TILE // STG):
        pltpu.sync_copy(rows0, acc.at[pl.ds(r0 + t * STG, STG), :])
    plsc.subcore_barrier()

    _edge_loop(m, src, dst, acc, srcv, dstv, rows0, rows1, sem0, sem1,
               wid, lambda dv, j: None)
    plsc.subcore_barrier()
    for t in range(ROWS_PER_TILE // STG):
        sl = pl.ds(r0 + t * STG, STG)
        pltpu.sync_copy(acc.at[sl, :], rows0)
        pltpu.sync_copy(rows0, part.at[c, sl, :])


def _agg_deg(m, src, dst, zrows, dzer, ones):
    return pl.kernel(
        _agg_deg_body,
        mesh=_sc_mesh(),
        out_type=[
            jax.ShapeDtypeStruct((NC, NPAD, D), jnp.float32),
            jax.ShapeDtypeStruct((NC, NPAD), jnp.float32),
        ],
        scratch_types=[
            pltpu.VMEM((HCHUNK, CHUNK), jnp.int32),
            pltpu.VMEM((HCHUNK, CHUNK), jnp.int32),
            pltpu.VMEM((CHUNK, D), jnp.float32),
            pltpu.VMEM((CHUNK, D), jnp.float32),
            pltpu.SemaphoreType.DMA,
            pltpu.SemaphoreType.DMA,
            pltpu.VMEM_SHARED((NPAD, D), jnp.float32),
            pltpu.VMEM((CHUNK,), jnp.float32),
            pltpu.VMEM((ROWS_PER_TILE,), jnp.float32),
            pltpu.VMEM_SHARED((NPAD,), jnp.float32),
        ],
    )(m, src, dst, zrows, dzer, ones)


def _agg(m, src, dst, zrows):
    return pl.kernel(
        _agg_body,
        mesh=_sc_mesh(),
        out_type=jax.ShapeDtypeStruct((NC, NPAD, D), jnp.float32),
        scratch_types=[
            pltpu.VMEM((HCHUNK, CHUNK), jnp.int32),
            pltpu.VMEM((HCHUNK, CHUNK), jnp.int32),
            pltpu.VMEM((CHUNK, D), jnp.float32),
            pltpu.VMEM((CHUNK, D), jnp.float32),
            pltpu.SemaphoreType.DMA,
            pltpu.SemaphoreType.DMA,
            pltpu.VMEM_SHARED((NPAD, D), jnp.float32),
        ],
    )(m, src, dst, zrows)


def _pair_gather_body(h1, h2, il, ir, L1, L2, R1, R2, idxv, rows, sem):
    c = lax.axis_index("c")
    s = lax.axis_index("s")
    wid = s * NC + c
    sl = pl.ds(wid * PPW, PPW)
    pltpu.sync_copy(il.at[sl], idxv)
    pltpu.async_copy(h1.at[idxv], rows, sem).wait()
    pltpu.sync_copy(rows, L1.at[sl, :])
    pltpu.async_copy(h2.at[idxv], rows, sem).wait()
    pltpu.sync_copy(rows, L2.at[sl, :])
    pltpu.sync_copy(ir.at[sl], idxv)
    pltpu.async_copy(h1.at[idxv], rows, sem).wait()
    pltpu.sync_copy(rows, R1.at[sl, :])
    pltpu.async_copy(h2.at[idxv], rows, sem).wait()
    pltpu.sync_copy(rows, R2.at[sl, :])


def _pair_gather(h1, h2, il, ir):
    return pl.kernel(
        _pair_gather_body,
        mesh=_sc_mesh(),
        out_type=[jax.ShapeDtypeStruct((B, D), jnp.float32)] * 4,
        scratch_types=[
            pltpu.VMEM((PPW,), jnp.int32),
            pltpu.VMEM((PPW, D), jnp.float32),
            pltpu.SemaphoreType.DMA,
        ],
    )(h1, h2, il, ir)


@jax.jit
def kernel(x, adj, idx, W1, b1, W2, b2):
    src = adj[0].astype(jnp.int32)
    dst = adj[1].astype(jnp.int32)
    il = idx[:, 0].astype(jnp.int32)
    ir = idx[:, 1].astype(jnp.int32)
    pad_e = EPAD - E
    src = jnp.concatenate([src, jnp.zeros((pad_e,), jnp.int32)])
    dst = jnp.concatenate([dst, jnp.full((pad_e,), NPAD - 1, jnp.int32)])
    src = src.reshape(EPAD // CHUNK, CHUNK)
    dst = dst.reshape(EPAD // CHUNK, CHUNK)
    xpad = jnp.pad(x, ((0, NPAD - N), (0, 0)))
    zrows = jnp.zeros((STG, D), jnp.float32)
    dzer = jnp.zeros((ROWS_PER_TILE,), jnp.float32)
    ones = jnp.ones((CHUNK,), jnp.float32)

    m1 = _stage1(xpad, W1, b1.reshape(1, D))
    part1, degp = _agg_deg(m1, src, dst, zrows, dzer, ones)
    degp3 = degp.reshape(NC, NPAD, 1)
    h1, m2 = _stage2(part1, degp3, W2, b2.reshape(1, D))
    part2 = _agg(m2, src, dst, zrows)
    h2 = _stage3(part2, degp3)
    L1, L2, R1, R2 = _pair_gather(h1, h2, il, ir)
    probs = _stage4(L1, R1, L2, R2)
    return probs.reshape(B)


# speedup vs baseline: 3.5859x; 3.5859x over previous
import jax
import jax.numpy as jnp
from jax import lax
from jax.experimental import pallas as pl
from jax.experimental.pallas import tpu as pltpu
from jax.experimental.pallas import tpu_sc as plsc

N = 10000
E = 320000
D = 128
B = 4096
R_FD = 2.0
T_FD = 1.0

NC = 2
NS = 16
NW = NC * NS
L = 16

NPAD = 10240
NRT = NPAD // NS
CH = 128
EPW = 10240
NCHW = EPW // CH
EPAD = EPW * NW
BPW = B // NW

RB = 1024
DB = 512


def _expmap0_proj(u):
    nrm = jnp.maximum(jnp.sqrt(jnp.sum(u * u, axis=1, keepdims=True)), 1e-6)
    x = jnp.tanh(nrm) * u / nrm
    n2 = jnp.maximum(jnp.sqrt(jnp.sum(x * x, axis=1, keepdims=True)), 1e-6)
    return x * jnp.minimum(1.0, (1.0 - 1e-5) / n2)


def _logmap0(x):
    nrm = jnp.maximum(jnp.sqrt(jnp.sum(x * x, axis=1, keepdims=True)), 1e-6)
    p = jnp.clip(nrm, 1e-6, 1.0 - 1e-5)
    return 0.5 * jnp.log((1.0 + p) / (1.0 - p)) * x / nrm


def _stage1_body(x_ref, w_ref, b_ref, o_ref):
    xh = _expmap0_proj(x_ref[...])
    h = _logmap0(xh)
    o_ref[...] = (
        jnp.dot(h, w_ref[...], preferred_element_type=jnp.float32) + b_ref[...]
    )


def _stage2_body(agg_ref, deg_ref, w_ref, b_ref, h1_ref, m2_ref):
    deg = jnp.maximum(deg_ref[0] + deg_ref[1], 1.0)
    agg = agg_ref[0] + agg_ref[1]
    agg = jnp.maximum(agg / deg[:, None], 0.0)
    h1 = _expmap0_proj(agg)
    h1_ref[...] = h1
    h = _logmap0(h1)
    m2_ref[...] = (
        jnp.dot(h, w_ref[...], preferred_element_type=jnp.float32) + b_ref[...]
    )


def _stage3_body(agg_ref, deg_ref, h2_ref):
    deg = jnp.maximum(deg_ref[0] + deg_ref[1], 1.0)
    h2_ref[...] = _expmap0_proj((agg_ref[0] + agg_ref[1]) / deg[:, None])


def _stage4_body(sq_ref, o_ref):
    dist = jnp.sum(sq_ref[...], axis=1, keepdims=True)
    o_ref[...] = 1.0 / (jnp.exp((dist - R_FD) / T_FD) + 1.0)


def _stage1(xp, w1, b1r):
    return pl.pallas_call(
        _stage1_body,
        grid=(NPAD // RB,),
        in_specs=[
            pl.BlockSpec((RB, D), lambda i: (i, 0)),
            pl.BlockSpec((D, D), lambda i: (0, 0)),
            pl.BlockSpec((1, D), lambda i: (0, 0)),
        ],
        out_specs=pl.BlockSpec((RB, D), lambda i: (i, 0)),
        out_shape=jax.ShapeDtypeStruct((NPAD, D), jnp.float32),
    )(xp, w1, b1r)


def _stage2(agg1, degp, w2, b2r):
    return pl.pallas_call(
        _stage2_body,
        grid=(NPAD // RB,),
        in_specs=[
            pl.BlockSpec((NC, RB, D), lambda i: (0, i, 0)),
            pl.BlockSpec((NC, RB), lambda i: (0, i)),
            pl.BlockSpec((D, D), lambda i: (0, 0)),
            pl.BlockSpec((1, D), lambda i: (0, 0)),
        ],
        out_specs=[
            pl.BlockSpec((RB, D), lambda i: (i, 0)),
            pl.BlockSpec((RB, D), lambda i: (i, 0)),
        ],
        out_shape=[
            jax.ShapeDtypeStruct((NPAD, D), jnp.float32),
            jax.ShapeDtypeStruct((NPAD, D), jnp.float32),
        ],
    )(agg1, degp, w2, b2r)


def _stage3(agg2, degp):
    return pl.pallas_call(
        _stage3_body,
        grid=(NPAD // RB,),
        in_specs=[
            pl.BlockSpec((NC, RB, D), lambda i: (0, i, 0)),
            pl.BlockSpec((NC, RB), lambda i: (0, i)),
        ],
        out_specs=pl.BlockSpec((RB, D), lambda i: (i, 0)),
        out_shape=jax.ShapeDtypeStruct((NPAD, D), jnp.float32),
    )(agg2, degp)


def _stage4(sq):
    return pl.pallas_call(
        _stage4_body,
        grid=(B // DB,),
        in_specs=[pl.BlockSpec((DB, D), lambda i: (i, 0))],
        out_specs=pl.BlockSpec((DB, 1), lambda i: (i, 0)),
        out_shape=jax.ShapeDtypeStruct((B, 1), jnp.float32),
    )(sq)


def _sc_mesh():
    return plsc.VectorSubcoreMesh(core_axis_name="c", subcore_axis_name="s")


SB = 16
NSB = NCHW // SB


def _zero_acc(z2d, z1d, buf0, zvec, acc, dega, s):
    pltpu.sync_copy(z2d, buf0)
    for i in range(NRT // CH):
        pltpu.sync_copy(buf0, acc.at[pl.ds(s * NRT + i * CH, CH)])
    if dega is not None:
        pltpu.sync_copy(z1d, zvec)
        pltpu.sync_copy(zvec, dega.at[pl.ds(s * NRT, NRT)])


def _edge_sweep(m, srcf, dstf, wid, srcidx, dstidx, acc, dega, onesv,
                buf0, buf1, sem_g0, sem_g1, sem_s0, sem_s1):
    def half(j, buf, sem_g, sem_s):
        pltpu.make_async_copy(m.at[srcidx.at[j]], buf, sem_g).wait()
        pltpu.async_copy(buf, acc.at[dstidx.at[j]], sem_s, add=True)
        if dega is not None:
            pltpu.sync_copy(onesv, dega.at[dstidx.at[j]], add=True)
        pltpu.make_async_copy(buf, acc.at[dstidx.at[j]], sem_s).wait()

    for q in range(NSB):
        pltpu.sync_copy(srcf.at[wid, pl.ds(q * SB, SB)], srcidx)
        pltpu.sync_copy(dstf.at[wid, pl.ds(q * SB, SB)], dstidx)
        pltpu.async_copy(m.at[srcidx.at[0]], buf0, sem_g0)

        def pair(t, carry):
            j = 2 * t
            pltpu.async_copy(m.at[srcidx.at[j + 1]], buf1, sem_g1)
            half(j, buf0, sem_g0, sem_s0)

            @pl.when(j + 2 < SB)
            def _():
                pltpu.async_copy(m.at[srcidx.at[j + 2]], buf0, sem_g0)

            half(j + 1, buf1, sem_g1, sem_s1)
            return carry

        lax.fori_loop(0, SB // 2, pair, 0)


def _writeback(acc, dega, out, degp, c, s):
    for i in range(NRT // CH):
        off = s * NRT + i * CH
        pltpu.sync_copy(acc.at[pl.ds(off, CH)], out.at[c, pl.ds(off, CH)])
    if dega is not None:
        pltpu.sync_copy(dega.at[pl.ds(s * NRT, NRT)],
                        degp.at[c, pl.ds(s * NRT, NRT)])


def _agg_deg_body(m, srcf, dstf, z2d, z1d, ones, out, degp,
                  srcidx, dstidx, buf0, buf1, zvec, onesv,
                  acc, dega, sem_g0, sem_g1, sem_s0, sem_s1):
    c = lax.axis_index("c")
    s = lax.axis_index("s")
    wid = s * NC + c
    pltpu.sync_copy(ones, onesv)
    _zero_acc(z2d, z1d, buf0, zvec, acc, dega, s)
    plsc.subcore_barrier()
    _edge_sweep(m, srcf, dstf, wid, srcidx, dstidx, acc, dega, onesv,
                buf0, buf1, sem_g0, sem_g1, sem_s0, sem_s1)
    plsc.subcore_barrier()
    _writeback(acc, dega, out, degp, c, s)


def _agg_body(m, srcf, dstf, z2d, out,
              srcidx, dstidx, buf0, buf1,
              acc, sem_g0, sem_g1, sem_s0, sem_s1):
    c = lax.axis_index("c")
    s = lax.axis_index("s")
    wid = s * NC + c
    _zero_acc(z2d, None, buf0, None, acc, None, s)
    plsc.subcore_barrier()
    _edge_sweep(m, srcf, dstf, wid, srcidx, dstidx, acc, None, None,
                buf0, buf1, sem_g0, sem_g1, sem_s0, sem_s1)
    plsc.subcore_barrier()
    _writeback(acc, None, out, None, c, s)


_AGG_SCRATCH = [
    pltpu.VMEM((SB, CH), jnp.int32),
    pltpu.VMEM((SB, CH), jnp.int32),
    pltpu.VMEM((CH, D), jnp.float32),
    pltpu.VMEM((CH, D), jnp.float32),
]

_AGG_SEMS = [pltpu.SemaphoreType.DMA] * 4


def _agg_deg(m, srcf, dstf, z2d, z1d, ones):
    return pl.kernel(
        _agg_deg_body,
        mesh=_sc_mesh(),
        out_type=[
            jax.ShapeDtypeStruct((NC, NPAD, D), jnp.float32),
            jax.ShapeDtypeStruct((NC, NPAD), jnp.float32),
        ],
        scratch_types=_AGG_SCRATCH + [
            pltpu.VMEM((NRT,), jnp.float32),
            pltpu.VMEM((CH,), jnp.float32),
            pltpu.VMEM_SHARED((NPAD, D), jnp.float32),
            pltpu.VMEM_SHARED((NPAD,), jnp.float32),
        ] + _AGG_SEMS,
    )(m, srcf, dstf, z2d, z1d, ones)


def _agg(m, srcf, dstf, z2d):
    return pl.kernel(
        _agg_body,
        mesh=_sc_mesh(),
        out_type=jax.ShapeDtypeStruct((NC, NPAD, D), jnp.float32),
        scratch_types=_AGG_SCRATCH + [
            pltpu.VMEM_SHARED((NPAD, D), jnp.float32),
        ] + _AGG_SEMS,
    )(m, srcf, dstf, z2d)


def _decoder_body(h1, h2, il, ir, out,
                  ilv, irv, a1, b1, a2, b2, sq,
                  sem0, sem1, sem2, sem3):
    c = lax.axis_index("c")
    s = lax.axis_index("s")
    wid = s * NC + c
    pltpu.sync_copy(il.at[wid], ilv)
    pltpu.sync_copy(ir.at[wid], irv)
    pltpu.async_copy(h1.at[ilv], a1, sem0)
    pltpu.async_copy(h1.at[irv], b1, sem1)
    pltpu.async_copy(h2.at[ilv], a2, sem2)
    pltpu.async_copy(h2.at[irv], b2, sem3)
    pltpu.make_async_copy(h1.at[ilv], a1, sem0).wait()
    pltpu.make_async_copy(h1.at[irv], b1, sem1).wait()
    pltpu.make_async_copy(h2.at[ilv], a2, sem2).wait()
    pltpu.make_async_copy(h2.at[irv], b2, sem3).wait()

    def row(r, carry):
        for k in range(D // L):
            sl = pl.ds(k * L, L)
            d1 = a1[r, sl] - b1[r, sl]
            d2 = a2[r, sl] - b2[r, sl]
            sq[r, sl] = d1 * d1 + d2 * d2
        return carry

    lax.fori_loop(0, BPW, row, 0)
    pltpu.sync_copy(sq, out.at[pl.ds(wid * BPW, BPW)])


def _decoder(h1, h2, il, ir):
    return pl.kernel(
        _decoder_body,
        mesh=_sc_mesh(),
        out_type=jax.ShapeDtypeStruct((B, D), jnp.float32),
        scratch_types=(
            [pltpu.VMEM((BPW,), jnp.int32)] * 2
            + [pltpu.VMEM((BPW, D), jnp.float32)] * 5
            + [pltpu.SemaphoreType.DMA] * 4
        ),
    )(h1, h2, il, ir)


@jax.jit
def kernel(x, adj, idx, W1, b1, W2, b2):
    src = adj[0].astype(jnp.int32)
    dst = adj[1].astype(jnp.int32)
    pad_e = EPAD - E
    pad_src = jnp.arange(pad_e, dtype=jnp.int32) % N
    pad_dst = N + jnp.arange(pad_e, dtype=jnp.int32) % (NPAD - N)
    srcf = jnp.concatenate([src, pad_src]).reshape(NW, NCHW, CH)
    dstf = jnp.concatenate([dst, pad_dst]).reshape(NW, NCHW, CH)
    xp = jnp.pad(x, ((0, NPAD - N), (0, 0)))
    z2d = jnp.zeros((CH, D), jnp.float32)
    z1d = jnp.zeros((NRT,), jnp.float32)
    ones = jnp.ones((CH,), jnp.float32)
    il = idx[:, 0].astype(jnp.int32).reshape(NW, BPW)
    ir = idx[:, 1].astype(jnp.int32).reshape(NW, BPW)

    m1 = _stage1(xp, W1, b1.reshape(1, D))
    agg1, degp = _agg_deg(m1, srcf, dstf, z2d, z1d, ones)
    h1, m2 = _stage2(agg1, degp, W2, b2.reshape(1, D))
    agg2 = _agg(m2, srcf, dstf, z2d)
    h2 = _stage3(agg2, degp)
    sq = _decoder(h1, h2, il, ir)
    return _stage4(sq).reshape(B)
